# baseline (speedup 1.0000x reference)
import functools

import jax
import jax.numpy as jnp
from jax import lax
from jax.experimental import pallas as pl
from jax.experimental.pallas import tpu as pltpu
from jax.experimental.pallas import tpu_sc as plsc

BATCH = 4
SEQ = 2048
HIDDEN = 1024
NUM_TYPES = 2
LANES = 16
NJ = HIDDEN // LANES

ROWS = BATCH * SEQ
NW = 32
ROWS_PER_W = ROWS // NW
S_PER_W = SEQ // NW
CHUNK = 16
NWIN = S_PER_W // CHUNK
NCHUNKS = NWIN * BATCH
NTOK = 4


def _emb_body(ids_hbm, ttids_hbm, table_hbm, tt2_hbm, pos_hbm, out_hbm,
              idx_all, tid_all, tok0, tok1, tok2, tok3, pos0, pos1, tt2_v,
              g0, g1, g2, g3, o0, o1, o2, o3, p0, p1):
    nc = lax.axis_size("c")
    wid = lax.axis_index("s") * nc + lax.axis_index("c")
    sbase = wid * S_PER_W

    toks = [tok0, tok1, tok2, tok3]
    gsems = [g0, g1, g2, g3]
    osems = [o0, o1, o2, o3]

    stage = []
    for b in range(BATCH):
        off = b * SEQ + sbase
        stage.append(pltpu.make_async_copy(
            ids_hbm.at[pl.ds(off, S_PER_W)],
            idx_all.at[pl.ds(b * S_PER_W, S_PER_W)], g0))
        stage.append(pltpu.make_async_copy(
            ttids_hbm.at[pl.ds(off, S_PER_W)],
            tid_all.at[pl.ds(b * S_PER_W, S_PER_W)], g1))
    stage.append(pltpu.make_async_copy(tt2_hbm, tt2_v, g2))
    for cp in stage:
        cp.start()
    pltpu.make_async_copy(pos_hbm.at[pl.ds(sbase, CHUNK)], pos0, p0).start()
    for cp in stage:
        cp.wait()

    def chunk_ioff(cur):
        b = lax.rem(cur, BATCH)
        t = cur // BATCH
        return b * S_PER_W + t * CHUNK

    def issue(cur, tokb, gsem):
        idxvec = idx_all[pl.ds(chunk_ioff(cur), CHUNK)]
        pltpu.make_async_copy(table_hbm.at[idxvec], tokb, gsem).start()

    def pos_start(t, posb, psem):
        pltpu.make_async_copy(pos_hbm.at[pl.ds(sbase + t * CHUNK, CHUNK)],
                              posb, psem).start()

    def pos_wait(posb, psem):
        pltpu.make_async_copy(pos_hbm.at[pl.ds(0, CHUNK)], posb, psem).wait()

    def out_wait(q):
        pltpu.make_async_copy(toks[q], out_hbm.at[pl.ds(0, CHUNK)],
                              osems[q]).wait()

    for j in range(NJ):
        dsl = pl.ds(j * LANES, LANES)
        tt2_v[1, dsl] = tt2_v[1, dsl] - tt2_v[0, dsl]

    issue(0, tok0, g0)
    issue(1, tok1, g1)

    def step(cur, k, posb):
        @pl.when(cur + 2 < NCHUNKS)
        def _():
            @pl.when(cur >= 2)
            def _():
                out_wait((k + 2) % NTOK)
            issue(cur + 2, toks[(k + 2) % NTOK], gsems[(k + 2) % NTOK])

        pltpu.make_async_copy(table_hbm.at[idx_all[pl.ds(0, CHUNK)]],
                              toks[k], gsems[k]).wait()

        tokq = toks[k]
        ttf = tid_all[pl.ds(chunk_ioff(cur), CHUNK)].astype(jnp.float32)
        fvecs = [
            ttf.at[jnp.full((LANES,), r, jnp.int32)].get(
                mode="promise_in_bounds") for r in range(CHUNK)
        ]

        @plsc.parallel_loop(0, NJ, step=1, unroll=3)
        def jbody(j):
            dsl = pl.ds(j * LANES, LANES)
            t0 = tt2_v[0, dsl]
            d1 = tt2_v[1, dsl]
            for r in range(CHUNK):
                tokq[r, dsl] = (tokq[r, dsl] + posb[r, dsl]
                                + (t0 + fvecs[r] * d1))

        b = lax.rem(cur, BATCH)
        t = cur // BATCH
        flat_off = b * SEQ + sbase + t * CHUNK
        pltpu.make_async_copy(tokq, out_hbm.at[pl.ds(flat_off, CHUNK)],
                              osems[k]).start()

    def win_body(i, acc):
        t0w = 2 * i
        pos_start(t0w + 1, pos1, p1)
        pos_wait(pos0, p0)
        for b in range(BATCH):
            cur = t0w * BATCH + b
            step(cur, b % NTOK, pos0)
        @pl.when(t0w + 2 < NWIN)
        def _():
            pos_start(t0w + 2, pos0, p0)

        pos_wait(pos1, p1)
        for b in range(BATCH):
            cur = (t0w + 1) * BATCH + b
            step(cur, b % NTOK, pos1)
        return acc

    lax.fori_loop(0, NWIN // 2, win_body, 0)

    for q in range(NTOK):
        out_wait(q)


@jax.jit
def _emb_call(ids, ttids, token_table, token_type_table, pos_enc):
    mesh = plsc.VectorSubcoreMesh(core_axis_name="c", subcore_axis_name="s")
    f = pl.kernel(
        _emb_body,
        mesh=mesh,
        out_type=jax.ShapeDtypeStruct((ROWS, HIDDEN), jnp.float32),
        scratch_types=[
            pltpu.VMEM((ROWS_PER_W,), jnp.int32),
            pltpu.VMEM((ROWS_PER_W,), jnp.int32),
            pltpu.VMEM((CHUNK, HIDDEN), jnp.float32),
            pltpu.VMEM((CHUNK, HIDDEN), jnp.float32),
            pltpu.VMEM((CHUNK, HIDDEN), jnp.float32),
            pltpu.VMEM((CHUNK, HIDDEN), jnp.float32),
            pltpu.VMEM((CHUNK, HIDDEN), jnp.float32),
            pltpu.VMEM((CHUNK, HIDDEN), jnp.float32),
            pltpu.VMEM((NUM_TYPES, HIDDEN), jnp.float32),
            pltpu.SemaphoreType.DMA,
            pltpu.SemaphoreType.DMA,
            pltpu.SemaphoreType.DMA,
            pltpu.SemaphoreType.DMA,
            pltpu.SemaphoreType.DMA,
            pltpu.SemaphoreType.DMA,
            pltpu.SemaphoreType.DMA,
            pltpu.SemaphoreType.DMA,
            pltpu.SemaphoreType.DMA,
            pltpu.SemaphoreType.DMA,
        ],
    )
    return f(ids, ttids, token_table, token_type_table, pos_enc)


def kernel(input_ids, token_type_ids, token_table, token_type_table, pos_enc):
    B, S = input_ids.shape
    ids = input_ids.reshape(-1).astype(jnp.int32)
    ttids = token_type_ids.reshape(-1).astype(jnp.int32)
    out = _emb_call(ids, ttids, token_table.astype(jnp.float32),
                    token_type_table.astype(jnp.float32),
                    pos_enc.astype(jnp.float32))
    return out.reshape(B, S, HIDDEN)


# speedup vs baseline: 1.1286x; 1.1286x over previous
import functools

import jax
import jax.numpy as jnp
from jax import lax
from jax.experimental import pallas as pl
from jax.experimental.pallas import tpu as pltpu
from jax.experimental.pallas import tpu_sc as plsc

BATCH = 4
SEQ = 2048
HIDDEN = 1024
NUM_TYPES = 2
LANES = 16
NJ = HIDDEN // LANES

ROWS = BATCH * SEQ
NW = 32
ROWS_PER_W = ROWS // NW
S_PER_W = SEQ // NW
CHUNK = 16
NWIN = S_PER_W // CHUNK
NCHUNKS = NWIN * BATCH
NTOK = 4


def _emb_body(ids_hbm, ttids_hbm, table_hbm, tt2_hbm, pos_hbm, out_hbm,
              idx_all, tid_all, tok0, tok1, tok2, tok3, pos0, pos1, tt2_v,
              g0, g1, g2, g3, o0, o1, o2, o3, p0, p1):
    nc = lax.axis_size("c")
    wid = lax.axis_index("s") * nc + lax.axis_index("c")
    sbase = wid * S_PER_W

    toks = [tok0, tok1, tok2, tok3]
    gsems = [g0, g1, g2, g3]
    osems = [o0, o1, o2, o3]

    stage = []
    for b in range(BATCH):
        off = b * SEQ + sbase
        stage.append(pltpu.make_async_copy(
            ids_hbm.at[pl.ds(off, S_PER_W)],
            idx_all.at[pl.ds(b * S_PER_W, S_PER_W)], g0))
        stage.append(pltpu.make_async_copy(
            ttids_hbm.at[pl.ds(off, S_PER_W)],
            tid_all.at[pl.ds(b * S_PER_W, S_PER_W)], g1))
    stage.append(pltpu.make_async_copy(tt2_hbm, tt2_v, g2))
    for cp in stage:
        cp.start()
    pltpu.make_async_copy(pos_hbm.at[pl.ds(sbase, CHUNK)], pos0, p0).start()
    for cp in stage:
        cp.wait()

    def chunk_ioff(cur):
        b = lax.rem(cur, BATCH)
        t = cur // BATCH
        return b * S_PER_W + t * CHUNK

    def issue(cur, tokb, gsem):
        idxvec = idx_all[pl.ds(chunk_ioff(cur), CHUNK)]
        pltpu.make_async_copy(table_hbm.at[idxvec], tokb, gsem).start()

    def pos_start(t, posb, psem):
        pltpu.make_async_copy(pos_hbm.at[pl.ds(sbase + t * CHUNK, CHUNK)],
                              posb, psem).start()

    def pos_wait(posb, psem):
        pltpu.make_async_copy(pos_hbm.at[pl.ds(0, CHUNK)], posb, psem).wait()

    def out_wait(q):
        pltpu.make_async_copy(toks[q], out_hbm.at[pl.ds(0, CHUNK)],
                              osems[q]).wait()

    for j in range(NJ):
        dsl = pl.ds(j * LANES, LANES)
        tt2_v[1, dsl] = tt2_v[1, dsl] - tt2_v[0, dsl]

    issue(0, tok0, g0)
    issue(1, tok1, g1)

    def step(cur, k, posb):
        @pl.when(cur + 2 < NCHUNKS)
        def _():
            @pl.when(cur >= 2)
            def _():
                out_wait((k + 2) % NTOK)
            issue(cur + 2, toks[(k + 2) % NTOK], gsems[(k + 2) % NTOK])

        tokq = toks[k]
        ttf = tid_all[pl.ds(chunk_ioff(cur), CHUNK)].astype(jnp.float32)
        fvecs = [
            ttf.at[jnp.full((LANES,), r, jnp.int32)].get(
                mode="promise_in_bounds") for r in range(CHUNK)
        ]

        pltpu.make_async_copy(table_hbm.at[idx_all[pl.ds(0, CHUNK)]],
                              toks[k], gsems[k]).wait()

        @plsc.parallel_loop(0, NJ, step=1, unroll=2)
        def jbody(j):
            dsl = pl.ds(j * LANES, LANES)
            t0 = tt2_v[0, dsl]
            d1 = tt2_v[1, dsl]
            for r in range(CHUNK):
                tokq[r, dsl] = (tokq[r, dsl] + posb[r, dsl]
                                + (t0 + fvecs[r] * d1))

        b = lax.rem(cur, BATCH)
        t = cur // BATCH
        flat_off = b * SEQ + sbase + t * CHUNK
        pltpu.make_async_copy(tokq, out_hbm.at[pl.ds(flat_off, CHUNK)],
                              osems[k]).start()

    def win_body(i, acc):
        t0w = 2 * i
        pos_start(t0w + 1, pos1, p1)
        pos_wait(pos0, p0)
        for b in range(BATCH):
            cur = t0w * BATCH + b
            step(cur, b % NTOK, pos0)
        @pl.when(t0w + 2 < NWIN)
        def _():
            pos_start(t0w + 2, pos0, p0)

        pos_wait(pos1, p1)
        for b in range(BATCH):
            cur = (t0w + 1) * BATCH + b
            step(cur, b % NTOK, pos1)
        return acc

    lax.fori_loop(0, NWIN // 2, win_body, 0)

    for q in range(NTOK):
        out_wait(q)


@jax.jit
def _emb_call(ids, ttids, token_table, token_type_table, pos_enc):
    mesh = plsc.VectorSubcoreMesh(core_axis_name="c", subcore_axis_name="s")
    f = pl.kernel(
        _emb_body,
        mesh=mesh,
        out_type=jax.ShapeDtypeStruct((ROWS, HIDDEN), jnp.float32),
        scratch_types=[
            pltpu.VMEM((ROWS_PER_W,), jnp.int32),
            pltpu.VMEM((ROWS_PER_W,), jnp.int32),
            pltpu.VMEM((CHUNK, HIDDEN), jnp.float32),
            pltpu.VMEM((CHUNK, HIDDEN), jnp.float32),
            pltpu.VMEM((CHUNK, HIDDEN), jnp.float32),
            pltpu.VMEM((CHUNK, HIDDEN), jnp.float32),
            pltpu.VMEM((CHUNK, HIDDEN), jnp.float32),
            pltpu.VMEM((CHUNK, HIDDEN), jnp.float32),
            pltpu.VMEM((NUM_TYPES, HIDDEN), jnp.float32),
            pltpu.SemaphoreType.DMA,
            pltpu.SemaphoreType.DMA,
            pltpu.SemaphoreType.DMA,
            pltpu.SemaphoreType.DMA,
            pltpu.SemaphoreType.DMA,
            pltpu.SemaphoreType.DMA,
            pltpu.SemaphoreType.DMA,
            pltpu.SemaphoreType.DMA,
            pltpu.SemaphoreType.DMA,
            pltpu.SemaphoreType.DMA,
        ],
    )
    return f(ids, ttids, token_table, token_type_table, pos_enc)


def kernel(input_ids, token_type_ids, token_table, token_type_table, pos_enc):
    B, S = input_ids.shape
    ids = input_ids.reshape(-1).astype(jnp.int32)
    ttids = token_type_ids.reshape(-1).astype(jnp.int32)
    out = _emb_call(ids, ttids, token_table.astype(jnp.float32),
                    token_type_table.astype(jnp.float32),
                    pos_enc.astype(jnp.float32))
    return out.reshape(B, S, HIDDEN)


# speedup vs baseline: 1.1296x; 1.0009x over previous
import jax
import jax.numpy as jnp
from jax import lax
from jax.experimental import pallas as pl
from jax.experimental.pallas import tpu as pltpu
from jax.experimental.pallas import tpu_sc as plsc

BATCH = 4
SEQ = 2048
HIDDEN = 1024
NUM_TYPES = 2
LANES = 16
NJ = HIDDEN // LANES

ROWS = BATCH * SEQ
NW = 32
ROWS_PER_W = ROWS // NW
S_PER_W = SEQ // NW
CHUNK = 16
NWIN = S_PER_W // CHUNK
NCHUNKS = NWIN * BATCH
NTOK = 4


def _emb_body(ids_hbm, ttids_hbm, table_hbm, tt2_hbm, pos_hbm, out_hbm,
              idx_all, tid_all, tok0, tok1, tok2, tok3, pos0, pos1, tt2_v,
              g0, g1, g2, g3, o0, o1, o2, o3, p0, p1):
    nc = lax.axis_size("c")
    wid = lax.axis_index("s") * nc + lax.axis_index("c")
    sbase = wid * S_PER_W

    toks = [tok0, tok1, tok2, tok3]
    gsems = [g0, g1, g2, g3]
    osems = [o0, o1, o2, o3]

    stage = []
    for b in range(BATCH):
        off = b * SEQ + sbase
        stage.append(pltpu.make_async_copy(
            ids_hbm.at[pl.ds(off, S_PER_W)],
            idx_all.at[pl.ds(b * S_PER_W, S_PER_W)], g0))
        stage.append(pltpu.make_async_copy(
            ttids_hbm.at[pl.ds(off, S_PER_W)],
            tid_all.at[pl.ds(b * S_PER_W, S_PER_W)], g1))
    stage.append(pltpu.make_async_copy(tt2_hbm, tt2_v, g2))
    for cp in stage:
        cp.start()
    pltpu.make_async_copy(pos_hbm.at[pl.ds(sbase, CHUNK)], pos0, p0).start()
    for cp in stage:
        cp.wait()

    def chunk_ioff(cur):
        b = lax.rem(cur, BATCH)
        t = cur // BATCH
        return b * S_PER_W + t * CHUNK

    def issue(cur, tokb, gsem):
        idxvec = idx_all[pl.ds(chunk_ioff(cur), CHUNK)]
        pltpu.make_async_copy(table_hbm.at[idxvec], tokb, gsem).start()

    def pos_start(t, posb, psem):
        pltpu.make_async_copy(pos_hbm.at[pl.ds(sbase + t * CHUNK, CHUNK)],
                              posb, psem).start()

    def pos_wait(posb, psem):
        pltpu.make_async_copy(pos_hbm.at[pl.ds(0, CHUNK)], posb, psem).wait()

    def out_wait(q):
        pltpu.make_async_copy(toks[q], out_hbm.at[pl.ds(0, CHUNK)],
                              osems[q]).wait()

    for j in range(NJ):
        dsl = pl.ds(j * LANES, LANES)
        tt2_v[1, dsl] = tt2_v[1, dsl] - tt2_v[0, dsl]

    issue(0, tok0, g0)
    issue(1, tok1, g1)

    def step(cur, k, posb):
        @pl.when(cur + 2 < NCHUNKS)
        def _():
            @pl.when(cur >= 2)
            def _():
                out_wait((k + 2) % NTOK)
            issue(cur + 2, toks[(k + 2) % NTOK], gsems[(k + 2) % NTOK])

        tokq = toks[k]
        ttf = tid_all[pl.ds(chunk_ioff(cur), CHUNK)].astype(jnp.float32)
        fvecs = [
            ttf.at[jnp.full((LANES,), r, jnp.int32)].get(
                mode="promise_in_bounds") for r in range(CHUNK)
        ]

        pltpu.make_async_copy(table_hbm.at[idx_all[pl.ds(0, CHUNK)]],
                              toks[k], gsems[k]).wait()

        @plsc.parallel_loop(0, NJ, step=1, unroll=2)
        def jbody(j):
            dsl = pl.ds(j * LANES, LANES)
            t0 = tt2_v[0, dsl]
            d1 = tt2_v[1, dsl]
            for r in range(CHUNK):
                tokq[r, dsl] = (tokq[r, dsl] + posb[r, dsl]
                                + (t0 + fvecs[r] * d1))

        b = lax.rem(cur, BATCH)
        t = cur // BATCH
        flat_off = b * SEQ + sbase + t * CHUNK
        pltpu.make_async_copy(tokq, out_hbm.at[pl.ds(flat_off, CHUNK)],
                              osems[k]).start()

    def win_body(i, acc):
        t0w = 2 * i
        pos_start(t0w + 1, pos1, p1)
        pos_wait(pos0, p0)
        for b in range(BATCH):
            cur = t0w * BATCH + b
            step(cur, b % NTOK, pos0)
        @pl.when(t0w + 2 < NWIN)
        def _():
            pos_start(t0w + 2, pos0, p0)

        pos_wait(pos1, p1)
        for b in range(BATCH):
            cur = (t0w + 1) * BATCH + b
            step(cur, b % NTOK, pos1)
        return acc

    lax.fori_loop(0, NWIN // 2, win_body, 0)

    for q in range(NTOK):
        out_wait(q)


@jax.jit
def _emb_call(ids, ttids, token_table, token_type_table, pos_enc):
    mesh = plsc.VectorSubcoreMesh(core_axis_name="c", subcore_axis_name="s")
    f = pl.kernel(
        _emb_body,
        mesh=mesh,
        out_type=jax.ShapeDtypeStruct((ROWS, HIDDEN), jnp.float32),
        scratch_types=[
            pltpu.VMEM((ROWS_PER_W,), jnp.int32),
            pltpu.VMEM((ROWS_PER_W,), jnp.int32),
            pltpu.VMEM((CHUNK, HIDDEN), jnp.float32),
            pltpu.VMEM((CHUNK, HIDDEN), jnp.float32),
            pltpu.VMEM((CHUNK, HIDDEN), jnp.float32),
            pltpu.VMEM((CHUNK, HIDDEN), jnp.float32),
            pltpu.VMEM((CHUNK, HIDDEN), jnp.float32),
            pltpu.VMEM((CHUNK, HIDDEN), jnp.float32),
            pltpu.VMEM((NUM_TYPES, HIDDEN), jnp.float32),
            pltpu.SemaphoreType.DMA,
            pltpu.SemaphoreType.DMA,
            pltpu.SemaphoreType.DMA,
            pltpu.SemaphoreType.DMA,
            pltpu.SemaphoreType.DMA,
            pltpu.SemaphoreType.DMA,
            pltpu.SemaphoreType.DMA,
            pltpu.SemaphoreType.DMA,
            pltpu.SemaphoreType.DMA,
            pltpu.SemaphoreType.DMA,
        ],
    )
    return f(ids, ttids, token_table, token_type_table, pos_enc)


def kernel(input_ids, token_type_ids, token_table, token_type_table, pos_enc):
    B, S = input_ids.shape
    ids = input_ids.reshape(-1).astype(jnp.int32)
    ttids = token_type_ids.reshape(-1).astype(jnp.int32)
    out = _emb_call(ids, ttids, token_table.astype(jnp.float32),
                    token_type_table.astype(jnp.float32),
                    pos_enc.astype(jnp.float32))
    return out.reshape(B, S, HIDDEN)


# speedup vs baseline: 1.1562x; 1.0236x over previous
import jax
import jax.numpy as jnp
from jax import lax
from jax.experimental import pallas as pl
from jax.experimental.pallas import tpu as pltpu
from jax.experimental.pallas import tpu_sc as plsc

BATCH = 4
SEQ = 2048
HIDDEN = 1024
NUM_TYPES = 2
LANES = 16
NJ = HIDDEN // LANES

ROWS = BATCH * SEQ
NW = 32
ROWS_PER_W = ROWS // NW
S_PER_W = SEQ // NW
CHUNK = 16
NWIN = S_PER_W // CHUNK
NCHUNKS = NWIN * BATCH
NTOK = 4


def _emb_body(ids_hbm, ttids_hbm, table_hbm, tt2_hbm, pos_hbm, out_hbm,
              idx_all, tid_all, tok0, tok1, tok2, tok3, pos0, pos1, tt2_v,
              g0, g1, g2, g3, o0, o1, o2, o3, p0, p1):
    nc = lax.axis_size("c")
    wid = lax.axis_index("s") * nc + lax.axis_index("c")
    sbase = wid * S_PER_W

    toks = [tok0, tok1, tok2, tok3]
    gsems = [g0, g1, g2, g3]
    osems = [o0, o1, o2, o3]

    ids0 = pltpu.make_async_copy(ids_hbm.at[pl.ds(sbase, S_PER_W)],
                                 idx_all.at[pl.ds(0, S_PER_W)], p1)
    ids0.start()
    stage = []
    for b in range(1, BATCH):
        off = b * SEQ + sbase
        stage.append(pltpu.make_async_copy(
            ids_hbm.at[pl.ds(off, S_PER_W)],
            idx_all.at[pl.ds(b * S_PER_W, S_PER_W)], g2))
    for b in range(BATCH):
        off = b * SEQ + sbase
        stage.append(pltpu.make_async_copy(
            ttids_hbm.at[pl.ds(off, S_PER_W)],
            tid_all.at[pl.ds(b * S_PER_W, S_PER_W)], g3))
    stage.append(pltpu.make_async_copy(tt2_hbm, tt2_v, o0))
    for cp in stage:
        cp.start()
    pltpu.make_async_copy(pos_hbm.at[pl.ds(sbase, CHUNK)], pos0, p0).start()

    def chunk_ioff(cur):
        b = lax.rem(cur, BATCH)
        t = cur // BATCH
        return b * S_PER_W + t * CHUNK

    def issue(cur, tokb, gsem):
        idxvec = idx_all[pl.ds(chunk_ioff(cur), CHUNK)]
        pltpu.make_async_copy(table_hbm.at[idxvec], tokb, gsem).start()

    def pos_start(t, posb, psem):
        pltpu.make_async_copy(pos_hbm.at[pl.ds(sbase + t * CHUNK, CHUNK)],
                              posb, psem).start()

    def pos_wait(posb, psem):
        pltpu.make_async_copy(pos_hbm.at[pl.ds(0, CHUNK)], posb, psem).wait()

    def out_wait(q):
        pltpu.make_async_copy(toks[q], out_hbm.at[pl.ds(0, CHUNK)],
                              osems[q]).wait()

    ids0.wait()
    issue(0, tok0, g0)
    for cp in stage:
        cp.wait()
    issue(1, tok1, g1)

    for j in range(NJ):
        dsl = pl.ds(j * LANES, LANES)
        tt2_v[1, dsl] = tt2_v[1, dsl] - tt2_v[0, dsl]

    def step(cur, k, posb):
        @pl.when(cur + 2 < NCHUNKS)
        def _():
            @pl.when(cur >= 2)
            def _():
                out_wait((k + 2) % NTOK)
            issue(cur + 2, toks[(k + 2) % NTOK], gsems[(k + 2) % NTOK])

        tokq = toks[k]
        ttf = tid_all[pl.ds(chunk_ioff(cur), CHUNK)].astype(jnp.float32)
        fvecs = [
            ttf.at[jnp.full((LANES,), r, jnp.int32)].get(
                mode="promise_in_bounds") for r in range(CHUNK)
        ]

        pltpu.make_async_copy(table_hbm.at[idx_all[pl.ds(0, CHUNK)]],
                              toks[k], gsems[k]).wait()

        @plsc.parallel_loop(0, NJ, step=1, unroll=2)
        def jbody(j):
            dsl = pl.ds(j * LANES, LANES)
            t0 = tt2_v[0, dsl]
            d1 = tt2_v[1, dsl]
            for r in range(CHUNK):
                tokq[r, dsl] = (tokq[r, dsl] + posb[r, dsl]
                                + (t0 + fvecs[r] * d1))

        b = lax.rem(cur, BATCH)
        t = cur // BATCH
        flat_off = b * SEQ + sbase + t * CHUNK
        pltpu.make_async_copy(tokq, out_hbm.at[pl.ds(flat_off, CHUNK)],
                              osems[k]).start()

    def win_body(i, acc):
        t0w = 2 * i
        pos_start(t0w + 1, pos1, p1)
        pos_wait(pos0, p0)
        for b in range(BATCH):
            cur = t0w * BATCH + b
            step(cur, b % NTOK, pos0)
        @pl.when(t0w + 2 < NWIN)
        def _():
            pos_start(t0w + 2, pos0, p0)

        pos_wait(pos1, p1)
        for b in range(BATCH):
            cur = (t0w + 1) * BATCH + b
            step(cur, b % NTOK, pos1)
        return acc

    lax.fori_loop(0, NWIN // 2, win_body, 0)

    for q in range(NTOK):
        out_wait(q)


@jax.jit
def _emb_call(ids, ttids, token_table, token_type_table, pos_enc):
    mesh = plsc.VectorSubcoreMesh(core_axis_name="c", subcore_axis_name="s")
    f = pl.kernel(
        _emb_body,
        mesh=mesh,
        out_type=jax.ShapeDtypeStruct((ROWS, HIDDEN), jnp.float32),
        scratch_types=[
            pltpu.VMEM((ROWS_PER_W,), jnp.int32),
            pltpu.VMEM((ROWS_PER_W,), jnp.int32),
            pltpu.VMEM((CHUNK, HIDDEN), jnp.float32),
            pltpu.VMEM((CHUNK, HIDDEN), jnp.float32),
            pltpu.VMEM((CHUNK, HIDDEN), jnp.float32),
            pltpu.VMEM((CHUNK, HIDDEN), jnp.float32),
            pltpu.VMEM((CHUNK, HIDDEN), jnp.float32),
            pltpu.VMEM((CHUNK, HIDDEN), jnp.float32),
            pltpu.VMEM((NUM_TYPES, HIDDEN), jnp.float32),
            pltpu.SemaphoreType.DMA,
            pltpu.SemaphoreType.DMA,
            pltpu.SemaphoreType.DMA,
            pltpu.SemaphoreType.DMA,
            pltpu.SemaphoreType.DMA,
            pltpu.SemaphoreType.DMA,
            pltpu.SemaphoreType.DMA,
            pltpu.SemaphoreType.DMA,
            pltpu.SemaphoreType.DMA,
            pltpu.SemaphoreType.DMA,
        ],
    )
    return f(ids, ttids, token_table, token_type_table, pos_enc)


def kernel(input_ids, token_type_ids, token_table, token_type_table, pos_enc):
    B, S = input_ids.shape
    ids = input_ids.reshape(-1).astype(jnp.int32)
    ttids = token_type_ids.reshape(-1).astype(jnp.int32)
    out = _emb_call(ids, ttids, token_table.astype(jnp.float32),
                    token_type_table.astype(jnp.float32),
                    pos_enc.astype(jnp.float32))
    return out.reshape(B, S, HIDDEN)


# speedup vs baseline: 1.1599x; 1.0032x over previous
import jax
import jax.numpy as jnp
from jax import lax
from jax.experimental import pallas as pl
from jax.experimental.pallas import tpu as pltpu
from jax.experimental.pallas import tpu_sc as plsc

BATCH = 4
SEQ = 2048
HIDDEN = 1024
NUM_TYPES = 2
LANES = 16
NJ = HIDDEN // LANES

ROWS = BATCH * SEQ
NW = 32
ROWS_PER_W = ROWS // NW
S_PER_W = SEQ // NW
CHUNK = 16
NWIN = S_PER_W // CHUNK
NCHUNKS = NWIN * BATCH
NTOK = 4


def _emb_body(ids_hbm, ttids_hbm, table_hbm, tt2_hbm, pos_hbm, out_hbm,
              idx_all, tid_all, tok0, tok1, tok2, tok3, pos0, pos1, tt2_v,
              g0, g1, g2, g3, o0, o1, o2, o3, p0, p1):
    nc = lax.axis_size("c")
    wid = lax.axis_index("s") * nc + lax.axis_index("c")
    sbase = wid * S_PER_W

    toks = [tok0, tok1, tok2, tok3]
    gsems = [g0, g1, g2, g3]
    osems = [o0, o1, o2, o3]

    ids0 = pltpu.make_async_copy(ids_hbm.at[pl.ds(sbase, S_PER_W)],
                                 idx_all.at[pl.ds(0, S_PER_W)], p1)
    ids0.start()
    ids1 = pltpu.make_async_copy(
        ids_hbm.at[pl.ds(SEQ + sbase, S_PER_W)],
        idx_all.at[pl.ds(S_PER_W, S_PER_W)], o1)
    ids1.start()
    stage = []
    for b in range(2, BATCH):
        off = b * SEQ + sbase
        stage.append(pltpu.make_async_copy(
            ids_hbm.at[pl.ds(off, S_PER_W)],
            idx_all.at[pl.ds(b * S_PER_W, S_PER_W)], g2))
    for b in range(BATCH):
        off = b * SEQ + sbase
        stage.append(pltpu.make_async_copy(
            ttids_hbm.at[pl.ds(off, S_PER_W)],
            tid_all.at[pl.ds(b * S_PER_W, S_PER_W)], g3))
    stage.append(pltpu.make_async_copy(tt2_hbm, tt2_v, o0))
    for cp in stage:
        cp.start()
    pltpu.make_async_copy(pos_hbm.at[pl.ds(sbase, CHUNK)], pos0, p0).start()

    def chunk_ioff(cur):
        b = lax.rem(cur, BATCH)
        t = cur // BATCH
        return b * S_PER_W + t * CHUNK

    def issue(cur, tokb, gsem):
        idxvec = idx_all[pl.ds(chunk_ioff(cur), CHUNK)]
        pltpu.make_async_copy(table_hbm.at[idxvec], tokb, gsem).start()

    def pos_start(t, posb, psem):
        pltpu.make_async_copy(pos_hbm.at[pl.ds(sbase + t * CHUNK, CHUNK)],
                              posb, psem).start()

    def pos_wait(posb, psem):
        pltpu.make_async_copy(pos_hbm.at[pl.ds(0, CHUNK)], posb, psem).wait()

    def out_wait(q):
        pltpu.make_async_copy(toks[q], out_hbm.at[pl.ds(0, CHUNK)],
                              osems[q]).wait()

    ids0.wait()
    issue(0, tok0, g0)
    ids1.wait()
    issue(1, tok1, g1)
    for cp in stage:
        cp.wait()

    for j in range(NJ):
        dsl = pl.ds(j * LANES, LANES)
        tt2_v[1, dsl] = tt2_v[1, dsl] - tt2_v[0, dsl]

    def step(cur, k, posb):
        @pl.when(cur + 2 < NCHUNKS)
        def _():
            @pl.when(cur >= 2)
            def _():
                out_wait((k + 2) % NTOK)
            issue(cur + 2, toks[(k + 2) % NTOK], gsems[(k + 2) % NTOK])

        tokq = toks[k]
        ttf = tid_all[pl.ds(chunk_ioff(cur), CHUNK)].astype(jnp.float32)
        fvecs = [
            ttf.at[jnp.full((LANES,), r, jnp.int32)].get(
                mode="promise_in_bounds") for r in range(CHUNK)
        ]

        pltpu.make_async_copy(table_hbm.at[idx_all[pl.ds(0, CHUNK)]],
                              toks[k], gsems[k]).wait()

        @plsc.parallel_loop(0, NJ, step=1, unroll=2)
        def jbody(j):
            dsl = pl.ds(j * LANES, LANES)
            t0 = tt2_v[0, dsl]
            d1 = tt2_v[1, dsl]
            for r in range(CHUNK):
                tokq[r, dsl] = (tokq[r, dsl] + posb[r, dsl]
                                + (t0 + fvecs[r] * d1))

        b = lax.rem(cur, BATCH)
        t = cur // BATCH
        flat_off = b * SEQ + sbase + t * CHUNK
        pltpu.make_async_copy(tokq, out_hbm.at[pl.ds(flat_off, CHUNK)],
                              osems[k]).start()

    def win_body(i, acc):
        t0w = 2 * i
        pos_start(t0w + 1, pos1, p1)
        pos_wait(pos0, p0)
        for b in range(BATCH):
            cur = t0w * BATCH + b
            step(cur, b % NTOK, pos0)
        @pl.when(t0w + 2 < NWIN)
        def _():
            pos_start(t0w + 2, pos0, p0)

        pos_wait(pos1, p1)
        for b in range(BATCH):
            cur = (t0w + 1) * BATCH + b
            step(cur, b % NTOK, pos1)
        return acc

    lax.fori_loop(0, NWIN // 2, win_body, 0)

    for q in range(NTOK):
        out_wait(q)


@jax.jit
def _emb_call(ids, ttids, token_table, token_type_table, pos_enc):
    mesh = plsc.VectorSubcoreMesh(core_axis_name="c", subcore_axis_name="s")
    f = pl.kernel(
        _emb_body,
        mesh=mesh,
        out_type=jax.ShapeDtypeStruct((ROWS, HIDDEN), jnp.float32),
        scratch_types=[
            pltpu.VMEM((ROWS_PER_W,), jnp.int32),
            pltpu.VMEM((ROWS_PER_W,), jnp.int32),
            pltpu.VMEM((CHUNK, HIDDEN), jnp.float32),
            pltpu.VMEM((CHUNK, HIDDEN), jnp.float32),
            pltpu.VMEM((CHUNK, HIDDEN), jnp.float32),
            pltpu.VMEM((CHUNK, HIDDEN), jnp.float32),
            pltpu.VMEM((CHUNK, HIDDEN), jnp.float32),
            pltpu.VMEM((CHUNK, HIDDEN), jnp.float32),
            pltpu.VMEM((NUM_TYPES, HIDDEN), jnp.float32),
            pltpu.SemaphoreType.DMA,
            pltpu.SemaphoreType.DMA,
            pltpu.SemaphoreType.DMA,
            pltpu.SemaphoreType.DMA,
            pltpu.SemaphoreType.DMA,
            pltpu.SemaphoreType.DMA,
            pltpu.SemaphoreType.DMA,
            pltpu.SemaphoreType.DMA,
            pltpu.SemaphoreType.DMA,
            pltpu.SemaphoreType.DMA,
        ],
    )
    return f(ids, ttids, token_table, token_type_table, pos_enc)


def kernel(input_ids, token_type_ids, token_table, token_type_table, pos_enc):
    B, S = input_ids.shape
    ids = input_ids.reshape(-1).astype(jnp.int32)
    ttids = token_type_ids.reshape(-1).astype(jnp.int32)
    out = _emb_call(ids, ttids, token_table.astype(jnp.float32),
                    token_type_table.astype(jnp.float32),
                    pos_enc.astype(jnp.float32))
    return out.reshape(B, S, HIDDEN)
